# baseline (speedup 1.0000x reference)
import functools

import jax
import jax.numpy as jnp
from jax import lax
from jax.experimental import pallas as pl
from jax.experimental.pallas import tpu as pltpu
from jax.experimental.pallas import tpu_sc as plsc

VOCAB = 1000000
EMBED_DIM = 64
BATCH = 4096
SEQ = 200
BASE = 10000.0
HALF = EMBED_DIM // 2

NC = 2
NS = 16
NW = NC * NS
BBLK = BATCH // NW

_mesh = plsc.VectorSubcoreMesh(core_axis_name="c", subcore_axis_name="s")


@functools.partial(
    pl.kernel,
    mesh=_mesh,
    out_type=jax.ShapeDtypeStruct((SEQ, EMBED_DIM // 8, BATCH // BBLK, 8, BBLK),
                                  jnp.float32),
    scratch_types=[
        pltpu.VMEM((SEQ, BBLK), jnp.int32),
        pltpu.VMEM((BBLK, EMBED_DIM), jnp.float32),
        pltpu.VMEM((BBLK, EMBED_DIM), jnp.float32),
        pltpu.VMEM((EMBED_DIM // 8, 8, BBLK + 1), jnp.float32),
        pltpu.VMEM((EMBED_DIM // 8, 8, BBLK + 1), jnp.float32),
        pltpu.VMEM((SEQ, EMBED_DIM), jnp.float32),
        pltpu.SemaphoreType.DMA,
        pltpu.SemaphoreType.DMA,
        pltpu.SemaphoreType.DMA,
        pltpu.SemaphoreType.DMA,
    ],
    compiler_params=pltpu.CompilerParams(use_tc_tiling_on_sc=False,
                                         needs_layout_passes=False),
)
def _rope_embed(xt_hbm, table_hbm, rope_hbm, out_hbm,
                idx_v, buf0, buf1, obuf0, obuf1, rope_v,
                gsem0, gsem1, osem0, osem1):
    wid = lax.axis_index("s") * NC + lax.axis_index("c")
    b0 = wid * BBLK

    pltpu.sync_copy(xt_hbm.at[:, pl.ds(b0, BBLK)], idx_v)
    pltpu.sync_copy(rope_hbm, rope_v)

    bufs = (buf0, buf1)
    gsems = (gsem0, gsem1)
    obufs = (obuf0, obuf1)
    osems = (osem0, osem1)

    def start_gather(l, buf, sem):
        pltpu.async_copy(table_hbm.at[idx_v.at[l]], buf, sem)

    def wait_gather(buf, sem):
        pltpu.make_async_copy(table_hbm.at[pl.ds(0, BBLK)], buf, sem).wait()

    def start_out(l, obuf, sem):
        pltpu.async_copy(obuf.at[:, :, pl.ds(0, BBLK)],
                         out_hbm.at[l, :, wid], sem)

    def wait_out(l, obuf, sem):
        pltpu.make_async_copy(obuf.at[:, :, pl.ds(0, BBLK)],
                              out_hbm.at[l, :, wid],
                              sem).wait()

    iota16 = lax.iota(jnp.int32, 16)
    rows = []
    for k in range(4):
        rk = iota16 + 16 * k
        rows.append((rk // 8, rk % 8))

    def rope_chunk(l, buf, obuf):
        c0 = rope_v[l, pl.ds(0, 16)]
        c1 = rope_v[l, pl.ds(16, 16)]
        s0 = rope_v[l, pl.ds(32, 16)]
        s1 = rope_v[l, pl.ds(48, 16)]

        @plsc.parallel_loop(0, BBLK, unroll=32)
        def _row(j):
            jv = jnp.full((16,), j, jnp.int32)
            ev0 = buf[j, pl.ds(0, 16)]
            ev1 = buf[j, pl.ds(16, 16)]
            od0 = buf[j, pl.ds(32, 16)]
            od1 = buf[j, pl.ds(48, 16)]
            plsc.store_scatter(obuf, [rows[0][0], rows[0][1], jv],
                               ev0 * c0 - od0 * s0)
            plsc.store_scatter(obuf, [rows[1][0], rows[1][1], jv],
                               ev1 * c1 - od1 * s1)
            plsc.store_scatter(obuf, [rows[2][0], rows[2][1], jv],
                               ev0 * s0 + od0 * c0)
            plsc.store_scatter(obuf, [rows[3][0], rows[3][1], jv],
                               ev1 * s1 + od1 * c1)

    start_gather(0, buf0, gsem0)

    def outer(g, carry):
        for p in range(2):
            l = 2 * g + p
            wait_gather(bufs[p], gsems[p])

            @pl.when(l + 1 < SEQ)
            def _():
                start_gather(l + 1, bufs[1 - p], gsems[1 - p])

            @pl.when(l >= 2)
            def _():
                wait_out(l - 2, obufs[p], osems[p])

            rope_chunk(l, bufs[p], obufs[p])
            start_out(l, obufs[p], osems[p])
        return carry

    lax.fori_loop(0, SEQ // 2, outer, 0)
    wait_out(SEQ - 2, obufs[0], osems[0])
    wait_out(SEQ - 1, obufs[1], osems[1])


_TB = 32768


def _detile_body(tt_ref, o_ref):
    o_ref[...] = jnp.concatenate(
        [tt_ref[...].T, jnp.zeros((_TB, EMBED_DIM), jnp.float32)], axis=1)


_detile = pl.pallas_call(
    _detile_body,
    grid=((VOCAB + _TB - 1) // _TB,),
    in_specs=[pl.BlockSpec((EMBED_DIM, _TB), lambda i: (0, i))],
    out_specs=pl.BlockSpec((_TB, 2 * EMBED_DIM), lambda i: (i, 0)),
    out_shape=jax.ShapeDtypeStruct((VOCAB, 2 * EMBED_DIM), jnp.float32),
)


def _rope_table():
    positions = jnp.arange(SEQ, dtype=jnp.float32)[:, None]
    freqs_indices = jnp.arange(HALF, dtype=jnp.float32)
    freqs = 1.0 / (BASE ** (freqs_indices / EMBED_DIM))
    angles = positions * freqs
    return jnp.concatenate([jnp.cos(angles), jnp.sin(angles)], axis=-1)


def _kernel_impl(x, table):
    xt = (x.astype(jnp.int32) * 2).T
    tab_pad = _detile(table.T).reshape(2 * VOCAB, EMBED_DIM)
    out5 = _rope_embed(xt, tab_pad, _rope_table())
    return jnp.transpose(out5, (2, 4, 0, 1, 3)).reshape(BATCH, SEQ, EMBED_DIM)


kernel = jax.jit(_kernel_impl)


# speedup vs baseline: 1.2093x; 1.2093x over previous
import functools

import jax
import jax.numpy as jnp
from jax import lax
from jax.experimental import pallas as pl
from jax.experimental.pallas import tpu as pltpu
from jax.experimental.pallas import tpu_sc as plsc

VOCAB = 1000000
EMBED_DIM = 64
BATCH = 4096
SEQ = 200
BASE = 10000.0
HALF = EMBED_DIM // 2

NC = 2
NS = 16
NW = NC * NS
BBLK = BATCH // NW

_mesh = plsc.VectorSubcoreMesh(core_axis_name="c", subcore_axis_name="s")


@functools.partial(
    pl.kernel,
    mesh=_mesh,
    out_type=jax.ShapeDtypeStruct((SEQ, EMBED_DIM // 8, BATCH // BBLK, 8, BBLK),
                                  jnp.float32),
    scratch_types=[
        pltpu.VMEM((SEQ, BBLK), jnp.int32),
        pltpu.VMEM((BBLK, EMBED_DIM), jnp.float32),
        pltpu.VMEM((BBLK, EMBED_DIM), jnp.float32),
        pltpu.VMEM((BBLK, EMBED_DIM), jnp.float32),
        pltpu.VMEM((BBLK, EMBED_DIM), jnp.float32),
        pltpu.VMEM((EMBED_DIM // 8, 8, BBLK + 1), jnp.float32),
        pltpu.VMEM((EMBED_DIM // 8, 8, BBLK + 1), jnp.float32),
        pltpu.VMEM((SEQ, EMBED_DIM), jnp.float32),
        pltpu.SemaphoreType.DMA,
        pltpu.SemaphoreType.DMA,
        pltpu.SemaphoreType.DMA,
        pltpu.SemaphoreType.DMA,
        pltpu.SemaphoreType.DMA,
        pltpu.SemaphoreType.DMA,
    ],
    compiler_params=pltpu.CompilerParams(use_tc_tiling_on_sc=False,
                                         needs_layout_passes=False),
)
def _rope_embed(xt_hbm, table_hbm, rope_hbm, out_hbm,
                idx_v, buf0, buf1, buf2, buf3, obuf0, obuf1, rope_v,
                gsem0, gsem1, gsem2, gsem3, osem0, osem1):
    wid = lax.axis_index("s") * NC + lax.axis_index("c")
    b0 = wid * BBLK

    pltpu.sync_copy(xt_hbm.at[:, pl.ds(b0, BBLK)], idx_v)
    pltpu.sync_copy(rope_hbm, rope_v)

    bufs = (buf0, buf1, buf2, buf3)
    gsems = (gsem0, gsem1, gsem2, gsem3)
    obufs = (obuf0, obuf1)
    osems = (osem0, osem1)

    def start_gather(l, buf, sem):
        pltpu.async_copy(table_hbm.at[idx_v.at[l]], buf, sem)

    def wait_gather(buf, sem):
        pltpu.make_async_copy(table_hbm.at[pl.ds(0, BBLK)], buf, sem).wait()

    def start_out(l, obuf, sem):
        pltpu.async_copy(obuf.at[:, :, pl.ds(0, BBLK)],
                         out_hbm.at[l, :, wid], sem)

    def wait_out(l, obuf, sem):
        pltpu.make_async_copy(obuf.at[:, :, pl.ds(0, BBLK)],
                              out_hbm.at[l, :, wid],
                              sem).wait()

    iota16 = lax.iota(jnp.int32, 16)
    rows = []
    for k in range(4):
        rk = iota16 + 16 * k
        rows.append((rk // 8, rk % 8))

    def rope_chunk(l, buf, obuf):
        c0 = rope_v[l, pl.ds(0, 16)]
        c1 = rope_v[l, pl.ds(16, 16)]
        s0 = rope_v[l, pl.ds(32, 16)]
        s1 = rope_v[l, pl.ds(48, 16)]

        @plsc.parallel_loop(0, BBLK, unroll=16)
        def _row(j):
            jv = jnp.full((16,), j, jnp.int32)
            ev0 = buf[j, pl.ds(0, 16)]
            ev1 = buf[j, pl.ds(16, 16)]
            od0 = buf[j, pl.ds(32, 16)]
            od1 = buf[j, pl.ds(48, 16)]
            plsc.store_scatter(obuf, [rows[0][0], rows[0][1], jv],
                               ev0 * c0 - od0 * s0)
            plsc.store_scatter(obuf, [rows[1][0], rows[1][1], jv],
                               ev1 * c1 - od1 * s1)
            plsc.store_scatter(obuf, [rows[2][0], rows[2][1], jv],
                               ev0 * s0 + od0 * c0)
            plsc.store_scatter(obuf, [rows[3][0], rows[3][1], jv],
                               ev1 * s1 + od1 * c1)

    for l0 in range(3):
        start_gather(l0, bufs[l0], gsems[l0])

    def outer(g, carry):
        for p in range(4):
            l = 4 * g + p
            wait_gather(bufs[p], gsems[p])

            @pl.when(l + 3 < SEQ)
            def _():
                start_gather(l + 3, bufs[(p + 3) % 4], gsems[(p + 3) % 4])

            @pl.when(l >= 2)
            def _():
                wait_out(l - 2, obufs[p % 2], osems[p % 2])

            rope_chunk(l, bufs[p], obufs[p % 2])
            start_out(l, obufs[p % 2], osems[p % 2])
        return carry

    lax.fori_loop(0, SEQ // 4, outer, 0)
    wait_out(SEQ - 2, obufs[0], osems[0])
    wait_out(SEQ - 1, obufs[1], osems[1])


_TB = 32768


def _detile_body(tt_ref, o_ref):
    o_ref[...] = jnp.concatenate(
        [tt_ref[...].T, jnp.zeros((_TB, EMBED_DIM), jnp.float32)], axis=1)


_detile = pl.pallas_call(
    _detile_body,
    grid=((VOCAB + _TB - 1) // _TB,),
    in_specs=[pl.BlockSpec((EMBED_DIM, _TB), lambda i: (0, i))],
    out_specs=pl.BlockSpec((_TB, 2 * EMBED_DIM), lambda i: (i, 0)),
    out_shape=jax.ShapeDtypeStruct((VOCAB, 2 * EMBED_DIM), jnp.float32),
)


def _rope_table():
    positions = jnp.arange(SEQ, dtype=jnp.float32)[:, None]
    freqs_indices = jnp.arange(HALF, dtype=jnp.float32)
    freqs = 1.0 / (BASE ** (freqs_indices / EMBED_DIM))
    angles = positions * freqs
    return jnp.concatenate([jnp.cos(angles), jnp.sin(angles)], axis=-1)


def _kernel_impl(x, table):
    xt = (x.astype(jnp.int32) * 2).T
    tab_pad = _detile(table.T).reshape(2 * VOCAB, EMBED_DIM)
    out5 = _rope_embed(xt, tab_pad, _rope_table())
    return jnp.transpose(out5, (2, 4, 0, 1, 3)).reshape(BATCH, SEQ, EMBED_DIM)


kernel = jax.jit(_kernel_impl)


# speedup vs baseline: 1.2974x; 1.0728x over previous
import functools

import jax
import jax.numpy as jnp
from jax import lax
from jax.experimental import pallas as pl
from jax.experimental.pallas import tpu as pltpu
from jax.experimental.pallas import tpu_sc as plsc

VOCAB = 1000000
EMBED_DIM = 64
BATCH = 4096
SEQ = 200
BASE = 10000.0
HALF = EMBED_DIM // 2

NC = 2
NS = 16
NW = NC * NS
BBLK = BATCH // NW

_mesh = plsc.VectorSubcoreMesh(core_axis_name="c", subcore_axis_name="s")


@functools.partial(
    pl.kernel,
    mesh=_mesh,
    out_type=jax.ShapeDtypeStruct((SEQ, EMBED_DIM // 8, BATCH // BBLK, 8, BBLK),
                                  jnp.float32),
    scratch_types=[
        pltpu.VMEM((SEQ, BBLK), jnp.int32),
        pltpu.VMEM((BBLK, EMBED_DIM), jnp.float32),
        pltpu.VMEM((BBLK, EMBED_DIM), jnp.float32),
        pltpu.VMEM((EMBED_DIM // 8, 8, BBLK + 1), jnp.float32),
        pltpu.VMEM((EMBED_DIM // 8, 8, BBLK + 1), jnp.float32),
        pltpu.VMEM((SEQ, EMBED_DIM), jnp.float32),
        pltpu.SemaphoreType.DMA,
        pltpu.SemaphoreType.DMA,
        pltpu.SemaphoreType.DMA,
        pltpu.SemaphoreType.DMA,
    ],
    compiler_params=pltpu.CompilerParams(use_tc_tiling_on_sc=False,
                                         needs_layout_passes=False),
)
def _rope_embed(xt_hbm, table_hbm, rope_hbm, out_hbm,
                idx_v, buf0, buf1, obuf0, obuf1, rope_v,
                gsem0, gsem1, osem0, osem1):
    wid = lax.axis_index("s") * NC + lax.axis_index("c")
    b0 = wid * BBLK

    pltpu.sync_copy(xt_hbm.at[:, pl.ds(b0, BBLK)], idx_v)
    pltpu.sync_copy(rope_hbm, rope_v)

    bufs = (buf0, buf1)
    gsems = (gsem0, gsem1)
    obufs = (obuf0, obuf1)
    osems = (osem0, osem1)

    def start_gather(l, buf, sem):
        pltpu.async_copy(table_hbm.at[idx_v.at[l]], buf, sem)

    def wait_gather(buf, sem):
        pltpu.make_async_copy(table_hbm.at[pl.ds(0, BBLK)], buf, sem).wait()

    def start_out(l, obuf, sem):
        pltpu.async_copy(obuf.at[:, :, pl.ds(0, BBLK)],
                         out_hbm.at[l, :, wid], sem)

    def wait_out(l, obuf, sem):
        pltpu.make_async_copy(obuf.at[:, :, pl.ds(0, BBLK)],
                              out_hbm.at[l, :, wid],
                              sem).wait()

    iota16 = lax.iota(jnp.int32, 16)
    rows = []
    for k in range(4):
        rk = iota16 + 16 * k
        rows.append((rk // 8, rk % 8))

    def rope_chunk(l, buf, obuf):
        c0 = rope_v[l, pl.ds(0, 16)]
        c1 = rope_v[l, pl.ds(16, 16)]
        s0 = rope_v[l, pl.ds(32, 16)]
        s1 = rope_v[l, pl.ds(48, 16)]

        @plsc.parallel_loop(0, BBLK, unroll=16)
        def _row(j):
            jv = jnp.full((16,), j, jnp.int32)
            ev0 = buf[j, pl.ds(0, 16)]
            ev1 = buf[j, pl.ds(16, 16)]
            od0 = buf[j, pl.ds(32, 16)]
            od1 = buf[j, pl.ds(48, 16)]
            plsc.store_scatter(obuf, [rows[0][0], rows[0][1], jv],
                               ev0 * c0 - od0 * s0)
            plsc.store_scatter(obuf, [rows[1][0], rows[1][1], jv],
                               ev1 * c1 - od1 * s1)
            plsc.store_scatter(obuf, [rows[2][0], rows[2][1], jv],
                               ev0 * s0 + od0 * c0)
            plsc.store_scatter(obuf, [rows[3][0], rows[3][1], jv],
                               ev1 * s1 + od1 * c1)

    start_gather(0, buf0, gsem0)

    def outer(g, carry):
        for p in range(2):
            l = 2 * g + p
            wait_gather(bufs[p], gsems[p])

            @pl.when(l + 1 < SEQ)
            def _():
                start_gather(l + 1, bufs[1 - p], gsems[1 - p])

            @pl.when(l >= 2)
            def _():
                wait_out(l - 2, obufs[p], osems[p])

            rope_chunk(l, bufs[p], obufs[p])
            start_out(l, obufs[p], osems[p])
        return carry

    lax.fori_loop(0, SEQ // 2, outer, 0)
    wait_out(SEQ - 2, obufs[0], osems[0])
    wait_out(SEQ - 1, obufs[1], osems[1])


_TB = 32768


def _detile_body(tt_ref, o_ref):
    o_ref[...] = jnp.concatenate(
        [tt_ref[...].T, jnp.zeros((_TB, EMBED_DIM), jnp.float32)], axis=1)


_detile = pl.pallas_call(
    _detile_body,
    grid=((VOCAB + _TB - 1) // _TB,),
    in_specs=[pl.BlockSpec((EMBED_DIM, _TB), lambda i: (0, i))],
    out_specs=pl.BlockSpec((_TB, 2 * EMBED_DIM), lambda i: (i, 0)),
    out_shape=jax.ShapeDtypeStruct((VOCAB, 2 * EMBED_DIM), jnp.float32),
)


def _rope_table():
    positions = jnp.arange(SEQ, dtype=jnp.float32)[:, None]
    freqs_indices = jnp.arange(HALF, dtype=jnp.float32)
    freqs = 1.0 / (BASE ** (freqs_indices / EMBED_DIM))
    angles = positions * freqs
    return jnp.concatenate([jnp.cos(angles), jnp.sin(angles)], axis=-1)


def _kernel_impl(x, table):
    xt = (x.astype(jnp.int32) * 2).T
    tab_pad = _detile(table.T).reshape(2 * VOCAB, EMBED_DIM)
    out5 = _rope_embed(xt, tab_pad, _rope_table())
    return jnp.transpose(out5, (2, 4, 0, 1, 3)).reshape(BATCH, SEQ, EMBED_DIM)


kernel = jax.jit(_kernel_impl)
